# baseline (speedup 1.0000x reference)
import functools

import jax
import jax.numpy as jnp
from jax import lax
from jax.experimental import pallas as pl
from jax.experimental.pallas import tpu as pltpu
from jax.experimental.pallas import tpu_sc as plsc

NUM_SPEAKERS = 1000
NUM_UTTER = 16384
EMBED_DIM = 128
BATCH = 16384

NC = 2
NS = 16
COLS = EMBED_DIM // NC
ROWS_PER_TILE = NUM_UTTER // NS
CHUNK = 128
NCHUNK = ROWS_PER_TILE // CHUNK
TABLE_ROWS = 1024
ZROWS = TABLE_ROWS // NS


def _sc_body(audio_hbm, init_idx_hbm, q_idx_hbm, out_hbm,
             rowbuf, zbuf, idxbuf, table, sems):
    c = lax.axis_index("c")
    s = lax.axis_index("s")
    c0 = c * COLS
    base = s * ROWS_PER_TILE
    idx_row0 = s * NCHUNK

    idx_cp = pltpu.async_copy(
        init_idx_hbm.at[pl.ds(idx_row0, NCHUNK)],
        idxbuf.at[pl.ds(0, NCHUNK)], sems.at[24])
    idx_cq = pltpu.async_copy(
        q_idx_hbm.at[pl.ds(idx_row0, NCHUNK)],
        idxbuf.at[pl.ds(NCHUNK, NCHUNK)], sems.at[25])
    loads = [
        pltpu.async_copy(
            audio_hbm.at[pl.ds(base + j * CHUNK, CHUNK), pl.ds(c0, COLS)],
            rowbuf.at[pl.ds(j * CHUNK, CHUNK)], sems.at[j])
        for j in range(NCHUNK)
    ]

    zvec = jnp.zeros((16,), jnp.float32)

    def _zero_row(r, _):
        for cc in range(COLS // 16):
            zbuf[r, pl.ds(cc * 16, 16)] = zvec
        return _

    lax.fori_loop(0, ZROWS, _zero_row, 0)
    pltpu.sync_copy(zbuf, table.at[pl.ds(s * ZROWS, ZROWS)])

    idx_cp.wait()
    plsc.subcore_barrier()

    scats = []
    for j in range(NCHUNK):
        loads[j].wait()
        scats.append(pltpu.async_copy(
            rowbuf.at[pl.ds(j * CHUNK, CHUNK)],
            table.at[idxbuf.at[j]], sems.at[16 + j], add=True))
    for cp in scats:
        cp.wait()
    idx_cq.wait()

    plsc.subcore_barrier()

    gathers = [
        pltpu.async_copy(table.at[idxbuf.at[NCHUNK + j]],
                         rowbuf.at[pl.ds(j * CHUNK, CHUNK)], sems.at[8 + j])
        for j in range(NCHUNK)
    ]
    writes = []
    for j in range(NCHUNK):
        gathers[j].wait()
        writes.append(pltpu.async_copy(
            rowbuf.at[pl.ds(j * CHUNK, CHUNK)],
            out_hbm.at[pl.ds(base + j * CHUNK, CHUNK), pl.ds(c0, COLS)],
            sems.at[26]))
    for cp in writes:
        cp.wait()


@jax.jit
def _sc_call(audio, i2d, q2d):
    mesh = plsc.VectorSubcoreMesh(core_axis_name="c", subcore_axis_name="s")
    f = functools.partial(
        pl.kernel,
        mesh=mesh,
        out_type=jax.ShapeDtypeStruct((BATCH, EMBED_DIM), jnp.float32),
        scratch_types=[
            pltpu.VMEM((ROWS_PER_TILE, COLS), jnp.float32),
            pltpu.VMEM((ZROWS, COLS), jnp.float32),
            pltpu.VMEM((2 * NCHUNK, CHUNK), jnp.int32),
            pltpu.VMEM_SHARED((TABLE_ROWS, COLS), jnp.float32),
            pltpu.SemaphoreType.DMA((27,)),
        ],
        compiler_params=pltpu.CompilerParams(use_tc_tiling_on_sc=False),
    )(_sc_body)
    return f(audio, i2d, q2d)


def kernel(speaker_ids, init_speaker_ids, audio_encodings):
    i2d = init_speaker_ids.astype(jnp.int32).reshape(NS * NCHUNK, CHUNK)
    q2d = speaker_ids.astype(jnp.int32).reshape(NS * NCHUNK, CHUNK)
    return _sc_call(audio_encodings, i2d, q2d)
